# Initial kernel scaffold; baseline (speedup 1.0000x reference)
import jax
import jax.numpy as jnp
from jax.experimental import pallas as pl


def kernel(x):
    raise NotImplementedError("write your pallas kernel here")


# speedup vs baseline: 3.3968x; 3.3968x over previous
import jax
import jax.numpy as jnp
import numpy as np
from jax.experimental import pallas as pl
from jax.experimental.pallas import tpu as pltpu

B, S, F = 4, 8192, 1024
G, GS, K = 16, 64, 256
SB = 2048


def _norms_body(x_ref, e_ref, n_ref):
    xb = x_ref[0]
    n_ref[0] = jnp.dot(xb * xb, e_ref[...], preferred_element_type=jnp.float32,
                       precision=jax.lax.Precision.HIGHEST)


def _select_body(n_ref, m_ref):
    v = jax.lax.bitcast_convert_type(n_ref[...], jnp.int32)

    def body(i, cur):
        cand = cur | jax.lax.shift_left(jnp.int32(1), 30 - i)
        cnt = jnp.sum((v >= cand).astype(jnp.int32), axis=1, keepdims=True)
        return jnp.where(cnt >= K, cand, cur)

    thr = jax.lax.fori_loop(0, 31, body, jnp.zeros((B * G, 1), jnp.int32))
    m_ref[...] = (v >= thr).astype(jnp.float32)


def _apply_body(x_ref, m_ref, et_ref, o_ref):
    mexp = jnp.dot(m_ref[0], et_ref[...], preferred_element_type=jnp.float32)
    o_ref[0] = x_ref[0] * mexp


def kernel(x):
    e = (jnp.arange(F, dtype=jnp.int32)[:, None] // GS
         == jnp.arange(G, dtype=jnp.int32)[None, :]).astype(jnp.float32)

    norms = pl.pallas_call(
        _norms_body,
        grid=(B, S // SB),
        in_specs=[
            pl.BlockSpec((1, SB, F), lambda i, j: (i, j, 0)),
            pl.BlockSpec((F, G), lambda i, j: (0, 0)),
        ],
        out_specs=pl.BlockSpec((1, SB, G), lambda i, j: (i, j, 0)),
        out_shape=jax.ShapeDtypeStruct((B, S, G), jnp.float32),
    )(x, e)

    nt = norms.transpose(0, 2, 1).reshape(B * G, S)

    mask_t = pl.pallas_call(
        _select_body,
        out_shape=jax.ShapeDtypeStruct((B * G, S), jnp.float32),
    )(nt)

    maskg = mask_t.reshape(B, G, S).transpose(0, 2, 1)

    out = pl.pallas_call(
        _apply_body,
        grid=(B, S // SB),
        in_specs=[
            pl.BlockSpec((1, SB, F), lambda i, j: (i, j, 0)),
            pl.BlockSpec((1, SB, G), lambda i, j: (i, j, 0)),
            pl.BlockSpec((G, F), lambda i, j: (0, 0)),
        ],
        out_specs=pl.BlockSpec((1, SB, F), lambda i, j: (i, j, 0)),
        out_shape=jax.ShapeDtypeStruct((B, S, F), jnp.float32),
    )(x, maskg, e.T)
    return out


# speedup vs baseline: 4.4232x; 1.3022x over previous
import jax
import jax.numpy as jnp
import numpy as np
from jax.experimental import pallas as pl
from jax.experimental.pallas import tpu as pltpu

B, S, F = 4, 8192, 1024
G, GS, K = 16, 64, 256
SB = 2048


def _norms_body(x_ref, e_ref, n_ref):
    xb = x_ref[0]
    xx = xb * xb
    eb = e_ref[...]
    h1 = xx.astype(jnp.bfloat16)
    r1 = xx - h1.astype(jnp.float32)
    h2 = r1.astype(jnp.bfloat16)
    h3 = (r1 - h2.astype(jnp.float32)).astype(jnp.bfloat16)
    acc = (jnp.dot(h1, eb, preferred_element_type=jnp.float32)
           + jnp.dot(h2, eb, preferred_element_type=jnp.float32)
           + jnp.dot(h3, eb, preferred_element_type=jnp.float32))
    n_ref[0] = acc


def _select_body(n_ref, m_ref):
    v = jax.lax.bitcast_convert_type(n_ref[...], jnp.int32)

    def body(i, cur):
        cand = cur | jax.lax.shift_left(jnp.int32(1), 30 - i)
        cnt = jnp.sum((v >= cand).astype(jnp.int32), axis=1, keepdims=True)
        return jnp.where(cnt >= K, cand, cur)

    thr = jax.lax.fori_loop(0, 31, body, jnp.zeros((B * G, 1), jnp.int32))
    m_ref[...] = (v >= thr).astype(jnp.float32)


def _apply_body(x_ref, m_ref, et_ref, o_ref):
    mexp = jnp.dot(m_ref[0], et_ref[...], preferred_element_type=jnp.float32)
    o_ref[0] = x_ref[0] * mexp


def kernel(x):
    e = (jnp.arange(F, dtype=jnp.int32)[:, None] // GS
         == jnp.arange(G, dtype=jnp.int32)[None, :]).astype(jnp.float32)

    norms = pl.pallas_call(
        _norms_body,
        grid=(B, S // SB),
        in_specs=[
            pl.BlockSpec((1, SB, F), lambda i, j: (i, j, 0)),
            pl.BlockSpec((F, G), lambda i, j: (0, 0)),
        ],
        out_specs=pl.BlockSpec((1, SB, G), lambda i, j: (i, j, 0)),
        out_shape=jax.ShapeDtypeStruct((B, S, G), jnp.float32),
    )(x, e.astype(jnp.bfloat16))

    nt = norms.transpose(0, 2, 1).reshape(B * G, S)

    mask_t = pl.pallas_call(
        _select_body,
        out_shape=jax.ShapeDtypeStruct((B * G, S), jnp.float32),
    )(nt)

    maskg = mask_t.reshape(B, G, S).transpose(0, 2, 1)

    out = pl.pallas_call(
        _apply_body,
        grid=(B, S // SB),
        in_specs=[
            pl.BlockSpec((1, SB, F), lambda i, j: (i, j, 0)),
            pl.BlockSpec((1, SB, G), lambda i, j: (i, j, 0)),
            pl.BlockSpec((G, F), lambda i, j: (0, 0)),
        ],
        out_specs=pl.BlockSpec((1, SB, F), lambda i, j: (i, j, 0)),
        out_shape=jax.ShapeDtypeStruct((B, S, F), jnp.float32),
    )(x, maskg, e.T)
    return out
